# baseline (speedup 1.0000x reference)
import jax
import jax.numpy as jnp
from jax.experimental import pallas as pl
from jax.experimental.pallas import tpu as pltpu

TOP_K = 8
N_EXPERTS = 64
HIDDEN = 4096

BT = 1024
NQ = 4
ROWS_Q = BT // NQ


def _copy(x_hbm, blk, buf, sems, q):
    return pltpu.make_async_copy(
        x_hbm.at[pl.ds(blk * BT + q * ROWS_Q, ROWS_Q), :],
        buf.at[pl.ds(q * ROWS_Q, ROWS_Q), :],
        sems.at[q],
    )


def _start(x_hbm, blk, buf, sems):
    for q in range(NQ):
        _copy(x_hbm, blk, buf, sems, q).start()


def _wait(x_hbm, blk, buf, sems):
    for q in range(NQ):
        _copy(x_hbm, blk, buf, sems, q).wait()


def _compute(xblk, w_all, idx_ref, w_ref):
    logits = jax.lax.dot_general(
        xblk,
        w_all,
        (((1,), (1,)), ((), ())),
        preferred_element_type=jnp.float32,
    )
    s = jax.nn.sigmoid(logits).T

    iota = jax.lax.broadcasted_iota(jnp.int32, (N_EXPERTS, BT), 0).astype(
        jnp.float32
    )
    vals = []
    idxs = []
    for _ in range(TOP_K):
        m = jnp.max(s, axis=0, keepdims=True)
        hit = s >= m
        idx = jnp.min(jnp.where(hit, iota, float(N_EXPERTS)), axis=0, keepdims=True)
        vals.append(m)
        idxs.append(idx)
        s = jnp.where(iota == idx, -1.0, s)

    topv = jnp.concatenate(vals, axis=0)
    topi = jnp.concatenate(idxs, axis=0)
    denom = jnp.sum(topv, axis=0, keepdims=True) + 1e-20
    idx_ref[...] = topi.T.astype(jnp.int32)
    w_ref[...] = (topv / denom).T


def _gate_kernel(x_hbm, w_in_ref, idx_ref, w_ref, xbuf0, xbuf1, sem0, sem1):
    i = pl.program_id(0)
    nb = pl.num_programs(0)
    even = jax.lax.rem(i, 2) == 0
    w_all = w_in_ref[...]

    @pl.when(i == 0)
    def _():
        _start(x_hbm, 0, xbuf0, sem0)

    @pl.when(jnp.logical_and(even, i + 1 < nb))
    def _():
        _start(x_hbm, i + 1, xbuf1, sem1)

    @pl.when(jnp.logical_and(jnp.logical_not(even), i + 1 < nb))
    def _():
        _start(x_hbm, i + 1, xbuf0, sem0)

    @pl.when(even)
    def _():
        _wait(x_hbm, i, xbuf0, sem0)
        _compute(xbuf0[...], w_all, idx_ref, w_ref)

    @pl.when(jnp.logical_not(even))
    def _():
        _wait(x_hbm, i, xbuf1, sem1)
        _compute(xbuf1[...], w_all, idx_ref, w_ref)


@jax.jit
def _gate(flat, w):
    n_tokens = flat.shape[0]
    grid = (n_tokens // BT,)
    return pl.pallas_call(
        _gate_kernel,
        grid=grid,
        in_specs=[
            pl.BlockSpec(memory_space=pltpu.MemorySpace.HBM),
            pl.BlockSpec((N_EXPERTS, HIDDEN), lambda i: (0, 0)),
        ],
        out_specs=[
            pl.BlockSpec((BT, TOP_K), lambda i: (i, 0)),
            pl.BlockSpec((BT, TOP_K), lambda i: (i, 0)),
        ],
        out_shape=[
            jax.ShapeDtypeStruct((n_tokens, TOP_K), jnp.int32),
            jax.ShapeDtypeStruct((n_tokens, TOP_K), jnp.float32),
        ],
        scratch_shapes=[
            pltpu.VMEM((BT, HIDDEN), jnp.float32),
            pltpu.VMEM((BT, HIDDEN), jnp.float32),
            pltpu.SemaphoreType.DMA((NQ,)),
            pltpu.SemaphoreType.DMA((NQ,)),
        ],
    )(flat, w)


def kernel(hidden_states, W):
    bsz, seq_len, h = hidden_states.shape
    flat = hidden_states.reshape(-1, h)
    topk_idx, topk_weight = _gate(flat, W)
    return (topk_idx, topk_weight)
